# baseline (speedup 1.0000x reference)
import functools

import jax
import jax.numpy as jnp
from jax.experimental import pallas as pl
from jax.experimental.pallas import tpu as pltpu

_M = 1024
_D = 256
_BLK = 512


def _vq_kernel(x_ref, emb_ref, embt_ref,
               q_ref, inds_ref, psum_ref, hist_ref):
    x = x_ref[...]
    embt = embt_ref[...]
    e2 = jnp.sum(embt * embt, axis=0, keepdims=True)
    x2 = jnp.sum(x * x, axis=1, keepdims=True)
    s = jnp.dot(x.astype(jnp.bfloat16), embt.astype(jnp.bfloat16),
                preferred_element_type=jnp.float32)
    dmap = -((e2 + x2) - 2.0 * s)

    m = jnp.max(dmap, axis=1, keepdims=True)
    iota = jax.lax.broadcasted_iota(jnp.int32, dmap.shape, 1)
    k = jnp.min(jnp.where(dmap == m, iota, _M), axis=1, keepdims=True)
    inds_ref[...] = k

    p = jnp.exp(dmap - m)
    probs = p / jnp.sum(p, axis=1, keepdims=True)
    psum_ref[...] = jnp.sum(probs, axis=0, keepdims=True)[None]

    oh = (iota == k).astype(jnp.float32)
    hist_ref[...] = jnp.sum(oh, axis=0, keepdims=True)[None]

    q_ref[...] = jnp.dot(oh.astype(jnp.bfloat16), emb_ref[...].astype(jnp.bfloat16),
                         preferred_element_type=jnp.float32)


def _finish_kernel(n_rows, psum_ref, hist_ref, cp_ref, pp_ref):
    inv_n = 1.0 / n_rows
    hp = jnp.sum(hist_ref[...], axis=0) * inv_n
    cp_ref[...] = -jnp.sum(hp * (jnp.log2(hp + 1e-10)), axis=1, keepdims=True)
    ap = jnp.sum(psum_ref[...], axis=0) * inv_n
    pp_ref[...] = -jnp.sum(ap * (jnp.log2(ap + 1e-10)), axis=1, keepdims=True)


def kernel(x, embedding):
    bsz, tsz, csz = x.shape
    n = bsz * tsz
    x_flat = x.reshape(n, csz)
    emb = embedding[0]
    embt = emb.T
    nblocks = n // _BLK

    q, inds, psum, hist = pl.pallas_call(
        _vq_kernel,
        grid=(nblocks,),
        in_specs=[
            pl.BlockSpec((_BLK, _D), lambda i: (i, 0)),
            pl.BlockSpec((_M, _D), lambda i: (0, 0)),
            pl.BlockSpec((_D, _M), lambda i: (0, 0)),
        ],
        out_specs=[
            pl.BlockSpec((_BLK, _D), lambda i: (i, 0)),
            pl.BlockSpec((_BLK, 1), lambda i: (i, 0)),
            pl.BlockSpec((1, 1, _M), lambda i: (i, 0, 0)),
            pl.BlockSpec((1, 1, _M), lambda i: (i, 0, 0)),
        ],
        out_shape=[
            jax.ShapeDtypeStruct((n, _D), jnp.float32),
            jax.ShapeDtypeStruct((n, 1), jnp.int32),
            jax.ShapeDtypeStruct((nblocks, 1, _M), jnp.float32),
            jax.ShapeDtypeStruct((nblocks, 1, _M), jnp.float32),
        ],
        compiler_params=pltpu.CompilerParams(
            dimension_semantics=("parallel",),
        ),
    )(x_flat, emb, embt)

    cp, pp = pl.pallas_call(
        functools.partial(_finish_kernel, float(n)),
        out_shape=[
            jax.ShapeDtypeStruct((1, 1), jnp.float32),
            jax.ShapeDtypeStruct((1, 1), jnp.float32),
        ],
    )(psum, hist)

    quantized = q.reshape(bsz, tsz, csz)
    quantization_inds = inds.reshape(bsz, tsz, 1)
    return (quantized, cp[0, 0], pp[0, 0], quantization_inds)
